# baseline (speedup 1.0000x reference)
import functools

import jax
import jax.numpy as jnp
from jax import lax
from jax.experimental import pallas as pl
from jax.experimental.pallas import tpu as pltpu
from jax.experimental.pallas import tpu_sc as plsc

_BINS = 30
_HBINS = 32
_L = 16
_NC = 2
_NS = 16
_NW = _NC * _NS
_QG = 4

_LN2 = 0.6931471805599453
_LOG2_COEF = (
    -3.7216296, 10.143928, -15.955576, 19.71584, -17.883608, 11.797779,
    -5.5984416, 1.8633448, -0.41319442, 0.054859888, -0.0033009734,
)


def _vlog(x):
    bits = plsc.bitcast(x, jnp.int32)
    e = ((bits >> 23) - 127).astype(jnp.float32)
    m = plsc.bitcast((bits & 0x007FFFFF) | 0x3F800000, jnp.float32)
    c = [jnp.float32(v) for v in _LOG2_COEF]
    m2 = m * m
    m4 = m2 * m2
    m8 = m4 * m4
    q0 = (c[0] + c[1] * m) + (c[2] + c[3] * m) * m2
    q1 = (c[4] + c[5] * m) + (c[6] + c[7] * m) * m2
    q2 = (c[8] + c[9] * m) + c[10] * m2
    acc = (q0 + q1 * m4) + q2 * m8
    return (e + acc) * jnp.float32(_LN2)


def _sc_hist(simmat, dtoks, qtoks):
    B, Q, D = simmat.shape
    b_per_w = B // _NW
    nchunk = D // _L
    ngrp = Q // _QG
    hwords = _QG * _L * _HBINS

    mesh = plsc.VectorSubcoreMesh(core_axis_name="c", subcore_axis_name="s")

    @functools.partial(
        pl.kernel,
        mesh=mesh,
        compiler_params=pltpu.CompilerParams(needs_layout_passes=False),
        out_type=jax.ShapeDtypeStruct((B, Q * _BINS), jnp.float32),
        scratch_types=[
            pltpu.VMEM((D,), jnp.int32),
            pltpu.VMEM((D,), jnp.float32),
            pltpu.VMEM((2, _QG, D), jnp.float32),
            pltpu.VMEM((hwords,), jnp.float32),
            pltpu.VMEM((Q * _BINS,), jnp.float32),
            pltpu.VMEM((_L,), jnp.int32),
            pltpu.SemaphoreType.DMA,
            pltpu.SemaphoreType.DMA,
            pltpu.SemaphoreType.DMA,
            pltpu.SemaphoreType.DMA,
        ],
    )
    def hist_kernel(simmat_hbm, dtoks_hbm, qtoks_hbm, out_hbm,
                    dbuf, wbuf, sbuf, hbuf, obuf, qbuf,
                    sem_s0, sem_s1, sem_d, sem_o):
        sem_s = (sem_s0, sem_s1)
        wid = lax.axis_index("s") * _NC + lax.axis_index("c")
        lane = lax.iota(jnp.int32, _L)
        row_off = [lane * _HBINS + jnp.int32(r * _L * _HBINS)
                   for r in range(_QG)]
        ones = jnp.full((_L,), 1.0, jnp.float32)
        zeros = jnp.full((_L,), 0.0, jnp.float32)
        b0 = wid * b_per_w
        nsteps = b_per_w * ngrp

        copies = [None, None]
        copies[0] = pltpu.async_copy(
            simmat_hbm.at[b0, pl.ds(0, _QG)], sbuf.at[0], sem_s[0])

        for bi in range(b_per_w):
            b = b0 + bi
            pltpu.async_copy(dtoks_hbm.at[b], dbuf, sem_d).wait()
            pltpu.async_copy(qtoks_hbm.at[b], qbuf, sem_d).wait()

            @plsc.parallel_loop(0, nchunk, 1, unroll=8)
            def wbody(i):
                t = dbuf[pl.ds(i * _L, _L)]
                wbuf[pl.ds(i * _L, _L)] = jnp.where(t != 0, ones, zeros)

            for g in range(ngrp):
                s = bi * ngrp + g
                cur = s % 2
                if s + 1 < nsteps:
                    nb = b0 + (s + 1) // ngrp
                    ng = (s + 1) % ngrp
                    copies[1 - cur] = pltpu.async_copy(
                        simmat_hbm.at[nb, pl.ds(ng * _QG, _QG)],
                        sbuf.at[1 - cur], sem_s[1 - cur])

                @plsc.parallel_loop(0, hwords // _L, 1, unroll=8)
                def zbody(i):
                    hbuf[pl.ds(i * _L, _L)] = zeros

                copies[cur].wait()

                @plsc.parallel_loop(0, nchunk, 1, unroll=3)
                def cbody(i, cur=cur):
                    w = wbuf[pl.ds(i * _L, _L)]
                    for r in range(_QG):
                        v = sbuf[cur, r, pl.ds(i * _L, _L)]
                        t = (v + jnp.float32(1.000001)) * jnp.float32(14.5)
                        bins = t.astype(jnp.int32)
                        plsc.addupdate_scatter(hbuf, [bins + row_off[r]], w)

                for r in range(_QG):
                    q = g * _QG + r
                    qmv = plsc.load_gather(
                        qbuf, [jnp.full((_L,), q, jnp.int32)])
                    qm = jnp.where(qmv != 0, ones, zeros)
                    rbase = r * _L * _HBINS
                    for h in range(2):
                        hb = h * (_BINS - _L)
                        terms = [hbuf[pl.ds(rbase + l * _HBINS + hb, _L)]
                                 for l in range(_L)]
                        while len(terms) > 1:
                            terms = [terms[i] + terms[i + 1]
                                     for i in range(0, len(terms), 2)]
                        res = _vlog(terms[0] * qm + jnp.float32(1e-5))
                        obuf[pl.ds(q * _BINS + hb, _L)] = res

            pltpu.async_copy(obuf, out_hbm.at[b], sem_o).wait()

    return hist_kernel(simmat, dtoks, qtoks)


def kernel(simmat, dtoks, qtoks):
    B, Q, _ = simmat.shape
    out = _sc_hist(simmat, dtoks.astype(jnp.int32), qtoks.astype(jnp.int32))
    return out.reshape(B, Q, _BINS)


# speedup vs baseline: 1.0223x; 1.0223x over previous
import functools

import jax
import jax.numpy as jnp
from jax import lax
from jax.experimental import pallas as pl
from jax.experimental.pallas import tpu as pltpu
from jax.experimental.pallas import tpu_sc as plsc

_BINS = 30
_HBINS = 32
_L = 16
_NC = 2
_NS = 16
_NW = _NC * _NS
_QG = 4

_LN2 = 0.6931471805599453
_LOG2_COEF = (
    -3.7216296, 10.143928, -15.955576, 19.71584, -17.883608, 11.797779,
    -5.5984416, 1.8633448, -0.41319442, 0.054859888, -0.0033009734,
)


def _vlog(x):
    bits = plsc.bitcast(x, jnp.int32)
    e = ((bits >> 23) - 127).astype(jnp.float32)
    m = plsc.bitcast((bits & 0x007FFFFF) | 0x3F800000, jnp.float32)
    c = [jnp.float32(v) for v in _LOG2_COEF]
    m2 = m * m
    m4 = m2 * m2
    m8 = m4 * m4
    q0 = (c[0] + c[1] * m) + (c[2] + c[3] * m) * m2
    q1 = (c[4] + c[5] * m) + (c[6] + c[7] * m) * m2
    q2 = (c[8] + c[9] * m) + c[10] * m2
    acc = (q0 + q1 * m4) + q2 * m8
    return (e + acc) * jnp.float32(_LN2)


def _sc_hist(simmat, dtoks, qtoks):
    B, Q, D = simmat.shape
    b_per_w = B // _NW
    nchunk = D // _L
    ngrp = Q // _QG
    hwords = _QG * _L * _HBINS

    mesh = plsc.VectorSubcoreMesh(core_axis_name="c", subcore_axis_name="s")

    @functools.partial(
        pl.kernel,
        mesh=mesh,
        compiler_params=pltpu.CompilerParams(needs_layout_passes=False),
        out_type=jax.ShapeDtypeStruct((B, Q * _HBINS), jnp.float32),
        scratch_types=[
            pltpu.VMEM((D,), jnp.int32),
            pltpu.VMEM((D,), jnp.float32),
            pltpu.VMEM((2, _QG, D), jnp.float32),
            pltpu.VMEM((hwords,), jnp.float32),
            pltpu.VMEM((2 * Q * _HBINS,), jnp.float32),
            pltpu.VMEM((_L,), jnp.int32),
            pltpu.SemaphoreType.DMA,
            pltpu.SemaphoreType.DMA,
            pltpu.SemaphoreType.DMA,
            pltpu.SemaphoreType.DMA,
        ],
    )
    def hist_kernel(simmat_hbm, dtoks_hbm, qtoks_hbm, out_hbm,
                    dbuf, wbuf, sbuf, hbuf, obuf, qbuf,
                    sem_s0, sem_s1, sem_d, sem_o):
        sem_s = (sem_s0, sem_s1)
        wid = lax.axis_index("s") * _NC + lax.axis_index("c")
        lane = lax.iota(jnp.int32, _L)
        row_off = [lane * _HBINS + jnp.int32(r * _L * _HBINS)
                   for r in range(_QG)]
        ones = jnp.full((_L,), 1.0, jnp.float32)
        zeros = jnp.full((_L,), 0.0, jnp.float32)
        b0 = wid * b_per_w
        nsteps = b_per_w * ngrp

        copies = [None, None]
        copies[0] = pltpu.async_copy(
            simmat_hbm.at[b0, pl.ds(0, _QG)], sbuf.at[0], sem_s[0])
        out_copies = [None] * b_per_w

        for bi in range(b_per_w):
            b = b0 + bi
            if bi >= 2:
                out_copies[bi - 2].wait()
            pltpu.async_copy(dtoks_hbm.at[b], dbuf, sem_d).wait()
            pltpu.async_copy(qtoks_hbm.at[b], qbuf, sem_d).wait()

            @plsc.parallel_loop(0, nchunk, 1, unroll=8)
            def wbody(i):
                t = dbuf[pl.ds(i * _L, _L)]
                wbuf[pl.ds(i * _L, _L)] = jnp.where(t != 0, ones, zeros)

            for g in range(ngrp):
                s = bi * ngrp + g
                cur = s % 2
                if s + 1 < nsteps:
                    nb = b0 + (s + 1) // ngrp
                    ng = (s + 1) % ngrp
                    copies[1 - cur] = pltpu.async_copy(
                        simmat_hbm.at[nb, pl.ds(ng * _QG, _QG)],
                        sbuf.at[1 - cur], sem_s[1 - cur])

                @plsc.parallel_loop(0, hwords // _L, 1, unroll=8)
                def zbody(i):
                    hbuf[pl.ds(i * _L, _L)] = zeros

                copies[cur].wait()

                @plsc.parallel_loop(0, nchunk, 1, unroll=2)
                def cbody(i, cur=cur):
                    w = wbuf[pl.ds(i * _L, _L)]
                    for r in range(_QG):
                        v = sbuf[cur, r, pl.ds(i * _L, _L)]
                        t = (v + jnp.float32(1.000001)) * jnp.float32(14.5)
                        bins = t.astype(jnp.int32)
                        plsc.addupdate_scatter(hbuf, [bins + row_off[r]], w)

                for r in range(_QG):
                    q = g * _QG + r
                    qmv = plsc.load_gather(
                        qbuf, [jnp.full((_L,), q, jnp.int32)])
                    qm = jnp.where(qmv != 0, ones, zeros)
                    rbase = r * _L * _HBINS
                    for h in range(2):
                        hb = h * (_BINS - _L)
                        terms = [hbuf[pl.ds(rbase + l * _HBINS + hb, _L)]
                                 for l in range(_L)]
                        while len(terms) > 1:
                            terms = [terms[i] + terms[i + 1]
                                     for i in range(0, len(terms), 2)]
                        res = _vlog(terms[0] * qm + jnp.float32(1e-5))
                        obuf[pl.ds((bi % 2) * Q * _HBINS + q * _HBINS + hb,
                                   _L)] = res

            out_copies[bi] = pltpu.async_copy(
                obuf.at[pl.ds((bi % 2) * Q * _HBINS, Q * _HBINS)],
                out_hbm.at[b], sem_o)

        for c in out_copies[-2:]:
            c.wait()

    return hist_kernel(simmat, dtoks, qtoks)


def kernel(simmat, dtoks, qtoks):
    B, Q, _ = simmat.shape
    out = _sc_hist(simmat, dtoks.astype(jnp.int32), qtoks.astype(jnp.int32))
    return out.reshape(B, Q, _HBINS)[:, :, :_BINS]
